# baseline (speedup 1.0000x reference)
import functools

import jax
import jax.numpy as jnp
from jax.experimental import pallas as pl

_NUM_AGENT = 2048
_EMB = 256
_IN = 2 * _EMB
_DEPTH = 2
_M = 1024


def _dhgn_block(h0_ref, a_ref, adj_ref, wagg_ref, bagg_ref, wfcra_ref,
                bfcra_ref, out_ref):
    h = h0_ref[...]
    for k in range(_DEPTH):
        adj = adj_ref[k]
        deg = jnp.clip(jnp.sum(adj, axis=-1, keepdims=True), 1e-6, None)
        agg = jax.lax.dot_general(
            adj, a_ref[k],
            dimension_numbers=(((1,), (0,)), ((), ())),
            preferred_element_type=jnp.float32)
        agg = agg / deg
        emb = jax.lax.dot_general(
            agg, wagg_ref[k],
            dimension_numbers=(((1,), (0,)), ((), ())),
            preferred_element_type=jnp.float32)
        emb = jnp.maximum(emb + bagg_ref[k][None, :], 0.0)
        acc = jax.lax.dot_general(
            emb, wfcra_ref[k, :_EMB],
            dimension_numbers=(((1,), (0,)), ((), ())),
            preferred_element_type=jnp.float32)
        acc += jax.lax.dot_general(
            h, wfcra_ref[k, _EMB:],
            dimension_numbers=(((1,), (0,)), ((), ())),
            preferred_element_type=jnp.float32)
        h = jnp.maximum(acc + bfcra_ref[k][None, :], 0.0)
    out_ref[...] = h


@jax.jit
def kernel(h0, a, adjacent_mat, W_agg, b_agg, W_fcra, b_fcra):
    grid = (_NUM_AGENT // _M,)
    return pl.pallas_call(
        _dhgn_block,
        grid=grid,
        in_specs=[
            pl.BlockSpec((_M, _EMB), lambda i: (i, 0)),
            pl.BlockSpec((_DEPTH, _NUM_AGENT, _IN), lambda i: (0, 0, 0)),
            pl.BlockSpec((_DEPTH, _M, _NUM_AGENT), lambda i: (0, i, 0)),
            pl.BlockSpec((_DEPTH, _IN, _EMB), lambda i: (0, 0, 0)),
            pl.BlockSpec((_DEPTH, _EMB), lambda i: (0, 0)),
            pl.BlockSpec((_DEPTH, _IN, _EMB), lambda i: (0, 0, 0)),
            pl.BlockSpec((_DEPTH, _EMB), lambda i: (0, 0)),
        ],
        out_specs=pl.BlockSpec((_M, _EMB), lambda i: (i, 0)),
        out_shape=jax.ShapeDtypeStruct((_NUM_AGENT, _EMB), jnp.float32),
    )(h0, a, adjacent_mat, W_agg, b_agg, W_fcra, b_fcra)


# speedup vs baseline: 1.0658x; 1.0658x over previous
import functools

import jax
import jax.numpy as jnp
from jax.experimental import pallas as pl

_NUM_AGENT = 2048
_EMB = 256
_IN = 2 * _EMB
_DEPTH = 2
_M = 512


def _dhgn_block(h0_ref, a_ref, adj_ref, wagg_ref, bagg_ref, wfcra_ref,
                bfcra_ref, out_ref):
    h = h0_ref[...]
    for k in range(_DEPTH):
        adj = adj_ref[k]
        deg = jnp.clip(jnp.sum(adj, axis=-1, keepdims=True), 1e-6, None)
        agg = jax.lax.dot_general(
            adj, a_ref[k],
            dimension_numbers=(((1,), (0,)), ((), ())),
            preferred_element_type=jnp.float32)
        agg = agg / deg
        emb = jax.lax.dot_general(
            agg, wagg_ref[k],
            dimension_numbers=(((1,), (0,)), ((), ())),
            preferred_element_type=jnp.float32)
        emb = jnp.maximum(emb + bagg_ref[k][None, :], 0.0)
        acc = jax.lax.dot_general(
            emb, wfcra_ref[k, :_EMB],
            dimension_numbers=(((1,), (0,)), ((), ())),
            preferred_element_type=jnp.float32)
        acc += jax.lax.dot_general(
            h, wfcra_ref[k, _EMB:],
            dimension_numbers=(((1,), (0,)), ((), ())),
            preferred_element_type=jnp.float32)
        h = jnp.maximum(acc + bfcra_ref[k][None, :], 0.0)
    out_ref[...] = h


@jax.jit
def kernel(h0, a, adjacent_mat, W_agg, b_agg, W_fcra, b_fcra):
    grid = (_NUM_AGENT // _M,)
    return pl.pallas_call(
        _dhgn_block,
        grid=grid,
        in_specs=[
            pl.BlockSpec((_M, _EMB), lambda i: (i, 0)),
            pl.BlockSpec((_DEPTH, _NUM_AGENT, _IN), lambda i: (0, 0, 0)),
            pl.BlockSpec((_DEPTH, _M, _NUM_AGENT), lambda i: (0, i, 0)),
            pl.BlockSpec((_DEPTH, _IN, _EMB), lambda i: (0, 0, 0)),
            pl.BlockSpec((_DEPTH, _EMB), lambda i: (0, 0)),
            pl.BlockSpec((_DEPTH, _IN, _EMB), lambda i: (0, 0, 0)),
            pl.BlockSpec((_DEPTH, _EMB), lambda i: (0, 0)),
        ],
        out_specs=pl.BlockSpec((_M, _EMB), lambda i: (i, 0)),
        out_shape=jax.ShapeDtypeStruct((_NUM_AGENT, _EMB), jnp.float32),
    )(h0, a, adjacent_mat, W_agg, b_agg, W_fcra, b_fcra)
